# baseline (speedup 1.0000x reference)
import jax
import jax.numpy as jnp
from jax import lax
from jax.experimental import pallas as pl
from jax.experimental.pallas import tpu as pltpu
from jax.experimental.pallas import tpu_sc as plsc

NC = 2
NS = 16
NW = NC * NS
LANES = 16

B = 16384
L = 50
D = 32
ROWS_PER_W = B // NW
BLK = 16
NBLK = ROWS_PER_W // BLK
IDX_PER_BLK = BLK * L
IDX_PER_W = ROWS_PER_W * L
IDXW = 128
IDX_CHUNK = 16
TSTRIDE = LANES + 1
TSIZE = (L * D - 1) * TSTRIDE + LANES


def _sc_body(idx_hbm, table_hbm, attn_hbm, out_hbm,
             attn_v, idxs_v, idx_all, emb_v0, emb_v1,
             t_v, scores_v, out_v, sem0, sem1):
    wid = lax.axis_index("s") * NC + lax.axis_index("c")
    row0 = wid * ROWS_PER_W

    pltpu.sync_copy(attn_hbm, attn_v)

    iota = lax.iota(jnp.int32, LANES)
    zeros16 = jnp.zeros((LANES,), jnp.int32)
    attn_vecs = [plsc.load_gather(attn_v, [iota + h * LANES, zeros16])
                 for h in range(D // LANES)]

    emb_bufs = (emb_v0, emb_v1)
    sems = (sem0, sem1)

    def stage_chunk(ch, _):
        pltpu.sync_copy(
            idx_hbm.at[pl.ds(row0 + ch * IDX_CHUNK, IDX_CHUNK)], idxs_v)
        fbase = ch * (IDX_CHUNK * L)

        def stage_row(r, _):
            rows = jnp.full((LANES,), r, jnp.int32)
            wb = fbase + r * L
            for k in range(0, L, LANES):
                mask = None if k + LANES <= L else iota < (L - k)
                v = plsc.load_gather(idxs_v, [rows, iota + k], mask=mask)
                plsc.store_scatter(idx_all, [iota + (wb + k)], v, mask=mask)
            return 0

        lax.fori_loop(0, IDX_CHUNK, stage_row, 0)
        return 0

    lax.fori_loop(0, ROWS_PER_W // IDX_CHUNK, stage_chunk, 0)

    def start_gather(blk, p):
        pltpu.async_copy(
            table_hbm.at[idx_all.at[pl.ds(blk * IDX_PER_BLK, IDX_PER_BLK)]],
            emb_bufs[p], sems[p])

    def compute_block(blk, p):
        emb_v = emb_bufs[p]
        base = row0 + blk * BLK
        with jax.named_scope("gwait"):
            pltpu.make_async_copy(
                table_hbm.at[idx_all.at[pl.ds(blk * IDX_PER_BLK, IDX_PER_BLK)]],
                emb_v, sems[p]).wait()

        def repack(l, _):
            lrow = jnp.full((LANES,), l, jnp.int32)
            wbase = l * (D * TSTRIDE)
            for dg in range(D // LANES):
                for r in range(LANES):
                    rows = lrow + r * L
                    cols = iota + dg * LANES
                    v = plsc.load_gather(emb_v, [rows, cols])
                    widx = iota * TSTRIDE + (wbase + dg * LANES * TSTRIDE + r)
                    plsc.store_scatter(t_v, [widx], v)
            return 0

        with jax.named_scope("repack"):
            lax.fori_loop(0, L, repack, 0)

        def p1(l, _):
            tbase = l * (D * TSTRIDE)
            parts = [jnp.zeros((LANES,), jnp.float32) for _ in range(4)]
            for d in range(D):
                v = plsc.load_gather(t_v, [iota + (tbase + d * TSTRIDE)])
                parts[d % 4] = parts[d % 4] + v * attn_vecs[d // LANES][d % LANES]
            scores_v[pl.ds(l * LANES, LANES)] = (parts[0] + parts[1]) + (
                parts[2] + parts[3])
            return 0

        with jax.named_scope("pass1"):
            lax.fori_loop(0, L, p1, 0)

        def pmax(l, m):
            return jnp.maximum(m, scores_v[pl.ds(l * LANES, LANES)])

        m = lax.fori_loop(0, L, pmax, jnp.full((LANES,), -jnp.inf, jnp.float32))

        def pexp(l, denom):
            e = jnp.exp(scores_v[pl.ds(l * LANES, LANES)] - m)
            scores_v[pl.ds(l * LANES, LANES)] = e
            return denom + e

        with jax.named_scope("softmax"):
            denom = lax.fori_loop(0, L, pexp, jnp.zeros((LANES,), jnp.float32))
        inv = 1.0 / denom

        for h in range(D // LANES):
            def p2(l, accs):
                tbase = l * (D * TSTRIDE)
                e = scores_v[pl.ds(l * LANES, LANES)]
                out = []
                for i, acc in enumerate(accs):
                    d = h * LANES + i
                    v = plsc.load_gather(t_v, [iota + (tbase + d * TSTRIDE)])
                    out.append(acc + v * e)
                return tuple(out)

            with jax.named_scope("pass2"):
                accs = lax.fori_loop(
                    0, L, p2,
                    tuple(jnp.zeros((LANES,), jnp.float32) for _ in range(LANES)))
            for i, acc in enumerate(accs):
                col = jnp.full((LANES,), h * LANES + i, jnp.int32)
                plsc.store_scatter(out_v, [iota, col], acc * inv)

        pltpu.sync_copy(out_v, out_hbm.at[pl.ds(base, BLK)])

    start_gather(0, 0)

    def pair_body(i, carry):
        for b in range(2):
            blk = 2 * i + b
            p = b
            nxt = blk + 1

            @pl.when(nxt < NBLK)
            def _():
                start_gather(nxt, 1 - p)

            compute_block(blk, p)
        return carry

    lax.fori_loop(0, NBLK // 2, pair_body, 0)


@jax.jit
def kernel(indices, table, attn_w):
    idx2d = jnp.pad(indices.astype(jnp.int32), ((0, 0), (0, IDXW - L)))

    mesh = plsc.VectorSubcoreMesh(core_axis_name="c", subcore_axis_name="s")
    f = pl.kernel(
        _sc_body,
        out_type=jax.ShapeDtypeStruct((B, D), jnp.float32),
        mesh=mesh,
        scratch_types=[
            pltpu.VMEM((D, 1), jnp.float32),
            pltpu.VMEM((IDX_CHUNK, IDXW), jnp.int32),
            pltpu.VMEM((IDX_PER_W,), jnp.int32),
            pltpu.VMEM((IDX_PER_BLK, D), jnp.float32),
            pltpu.VMEM((IDX_PER_BLK, D), jnp.float32),
            pltpu.VMEM((TSIZE,), jnp.float32),
            pltpu.VMEM((L * LANES,), jnp.float32),
            pltpu.VMEM((BLK, D), jnp.float32),
            pltpu.SemaphoreType.DMA,
            pltpu.SemaphoreType.DMA,
        ],
        compiler_params=pltpu.CompilerParams(
            needs_layout_passes=False, use_tc_tiling_on_sc=False),
    )
    return f(idx2d, table, attn_w)


# speedup vs baseline: 1.0277x; 1.0277x over previous
import jax
import jax.numpy as jnp
from jax import lax
from jax.experimental import pallas as pl
from jax.experimental.pallas import tpu as pltpu
from jax.experimental.pallas import tpu_sc as plsc

NC = 2
NS = 16
NW = NC * NS
LANES = 16

B = 16384
L = 50
D = 32
ROWS_PER_W = B // NW
BLK = 16
NBLK = ROWS_PER_W // BLK
IDX_PER_BLK = BLK * L
IDX_PER_W = ROWS_PER_W * L
IDXW = 128
IDX_ROWS_PER_W = IDX_PER_W // IDXW
IDX_CHUNK = 25
TSTRIDE = LANES + 1
TSIZE = (L * D - 1) * TSTRIDE + LANES


def _sc_body(idx_hbm, table_hbm, attn_hbm, out_hbm,
             attn_v, idxs_v, idx_all, emb_v0, emb_v1,
             t_v, scores_v, out_v, sem0, sem1):
    wid = lax.axis_index("s") * NC + lax.axis_index("c")
    row0 = wid * ROWS_PER_W

    pltpu.sync_copy(attn_hbm, attn_v)

    iota = lax.iota(jnp.int32, LANES)
    zeros16 = jnp.zeros((LANES,), jnp.int32)
    attn_vecs = [plsc.load_gather(attn_v, [iota + h * LANES, zeros16])
                 for h in range(D // LANES)]

    emb_bufs = (emb_v0, emb_v1)
    sems = (sem0, sem1)

    idxrow0 = wid * IDX_ROWS_PER_W

    def stage_chunk(ch, _):
        pltpu.sync_copy(
            idx_hbm.at[pl.ds(idxrow0 + ch * IDX_CHUNK, IDX_CHUNK)], idxs_v)
        fbase = ch * (IDX_CHUNK * IDXW)

        def stage_row(r, _):
            rows = jnp.full((LANES,), r, jnp.int32)
            wb = fbase + r * IDXW
            for k in range(IDXW // LANES):
                v = plsc.load_gather(idxs_v, [rows, iota + k * LANES])
                plsc.store_scatter(idx_all, [iota + (wb + k * LANES)], v)
            return 0

        lax.fori_loop(0, IDX_CHUNK, stage_row, 0)
        return 0

    lax.fori_loop(0, IDX_ROWS_PER_W // IDX_CHUNK, stage_chunk, 0)

    def start_gather(blk, p):
        pltpu.async_copy(
            table_hbm.at[idx_all.at[pl.ds(blk * IDX_PER_BLK, IDX_PER_BLK)]],
            emb_bufs[p], sems[p])

    def compute_block(blk, p):
        emb_v = emb_bufs[p]
        base = row0 + blk * BLK
        with jax.named_scope("gwait"):
            pltpu.make_async_copy(
                table_hbm.at[idx_all.at[pl.ds(blk * IDX_PER_BLK, IDX_PER_BLK)]],
                emb_v, sems[p]).wait()

        def repack(l, _):
            lrow = jnp.full((LANES,), l, jnp.int32)
            wbase = l * (D * TSTRIDE)
            for dg in range(D // LANES):
                for r in range(LANES):
                    rows = lrow + r * L
                    cols = iota + dg * LANES
                    v = plsc.load_gather(emb_v, [rows, cols])
                    widx = iota * TSTRIDE + (wbase + dg * LANES * TSTRIDE + r)
                    plsc.store_scatter(t_v, [widx], v)
            return 0

        with jax.named_scope("repack"):
            lax.fori_loop(0, L, repack, 0)

        def p1(l, _):
            tbase = l * (D * TSTRIDE)
            parts = [jnp.zeros((LANES,), jnp.float32) for _ in range(4)]
            for d in range(D):
                v = plsc.load_gather(t_v, [iota + (tbase + d * TSTRIDE)])
                parts[d % 4] = parts[d % 4] + v * attn_vecs[d // LANES][d % LANES]
            scores_v[pl.ds(l * LANES, LANES)] = (parts[0] + parts[1]) + (
                parts[2] + parts[3])
            return 0

        with jax.named_scope("pass1"):
            lax.fori_loop(0, L, p1, 0)

        def pmax(l, m):
            return jnp.maximum(m, scores_v[pl.ds(l * LANES, LANES)])

        m = lax.fori_loop(0, L, pmax, jnp.full((LANES,), -jnp.inf, jnp.float32))

        def pexp(l, denom):
            e = jnp.exp(scores_v[pl.ds(l * LANES, LANES)] - m)
            scores_v[pl.ds(l * LANES, LANES)] = e
            return denom + e

        with jax.named_scope("softmax"):
            denom = lax.fori_loop(0, L, pexp, jnp.zeros((LANES,), jnp.float32))
        inv = 1.0 / denom

        for h in range(D // LANES):
            def p2(l, accs):
                tbase = l * (D * TSTRIDE)
                e = scores_v[pl.ds(l * LANES, LANES)]
                out = []
                for i, acc in enumerate(accs):
                    d = h * LANES + i
                    v = plsc.load_gather(t_v, [iota + (tbase + d * TSTRIDE)])
                    out.append(acc + v * e)
                return tuple(out)

            with jax.named_scope("pass2"):
                accs = lax.fori_loop(
                    0, L, p2,
                    tuple(jnp.zeros((LANES,), jnp.float32) for _ in range(LANES)))
            for i, acc in enumerate(accs):
                col = jnp.full((LANES,), h * LANES + i, jnp.int32)
                plsc.store_scatter(out_v, [iota, col], acc * inv)

        pltpu.sync_copy(out_v, out_hbm.at[pl.ds(base, BLK)])

    start_gather(0, 0)

    def pair_body(i, carry):
        for b in range(2):
            blk = 2 * i + b
            p = b
            nxt = blk + 1

            @pl.when(nxt < NBLK)
            def _():
                start_gather(nxt, 1 - p)

            compute_block(blk, p)
        return carry

    lax.fori_loop(0, NBLK // 2, pair_body, 0)


@jax.jit
def kernel(indices, table, attn_w):
    idx2d = indices.astype(jnp.int32).reshape(B * L // IDXW, IDXW)
    idx2d = lax.optimization_barrier(idx2d)

    mesh = plsc.VectorSubcoreMesh(core_axis_name="c", subcore_axis_name="s")
    f = pl.kernel(
        _sc_body,
        out_type=jax.ShapeDtypeStruct((B, D), jnp.float32),
        mesh=mesh,
        scratch_types=[
            pltpu.VMEM((D, 1), jnp.float32),
            pltpu.VMEM((IDX_CHUNK, IDXW), jnp.int32),
            pltpu.VMEM((IDX_PER_W,), jnp.int32),
            pltpu.VMEM((IDX_PER_BLK, D), jnp.float32),
            pltpu.VMEM((IDX_PER_BLK, D), jnp.float32),
            pltpu.VMEM((TSIZE,), jnp.float32),
            pltpu.VMEM((L * LANES,), jnp.float32),
            pltpu.VMEM((BLK, D), jnp.float32),
            pltpu.SemaphoreType.DMA,
            pltpu.SemaphoreType.DMA,
        ],
        compiler_params=pltpu.CompilerParams(
            needs_layout_passes=False, use_tc_tiling_on_sc=False),
    )
    return f(idx2d, table, attn_w)
